# baseline (speedup 1.0000x reference)
import functools

import jax
import jax.numpy as jnp
from jax import lax
from jax.experimental import pallas as pl
from jax.experimental.pallas import tpu as pltpu
from jax.experimental.pallas import tpu_sc as plsc

_NE = 16
_RK = 16
_HID = 256

_TB_GATE = 1024
_TB_MAIN = 2048

_SC_WORKERS = 32


def _gate_body(x_ref, w1_ref, b1_ref, w2_ref, out_ref):
    xb = x_ref[...].astype(jnp.bfloat16)
    h = jnp.tanh(
        jnp.dot(xb, w1_ref[...], preferred_element_type=jnp.float32) + b1_ref[...]
    )
    out_ref[...] = jnp.dot(
        h.astype(jnp.bfloat16), w2_ref[...], preferred_element_type=jnp.float32
    )


def _route_body(logits_hbm, comb_hbm, in_v, out_v):
    rows = in_v.shape[0]
    wid = lax.axis_index("c") * 16 + lax.axis_index("s")
    base = wid * rows
    pltpu.sync_copy(logits_hbm.at[pl.ds(base, rows)], in_v)

    @pl.loop(0, rows)
    def _(i):
        v = in_v[i]
        idx = lax.iota(jnp.int32, 16)
        m1 = jnp.max(v)
        i1 = jnp.min(jnp.where(v == m1, idx, 16))
        vm = jnp.where(idx == i1, -jnp.inf, v)
        m2 = jnp.max(vm)
        i2 = jnp.min(jnp.where(vm == m2, idx, 16))
        sel = (idx == i1) | (idx == i2)
        ex = jnp.exp(v - m1)
        num = jnp.where(sel, ex, 0.0)
        out_v[i] = num / jnp.sum(num)

    pltpu.sync_copy(out_v, comb_hbm.at[pl.ds(base, rows)])


def _main_body(x_ref, W_ref, b_ref, A_ref, B_ref, c_ref, M_ref, out_ref):
    xb = x_ref[...].astype(jnp.bfloat16)
    acc = jnp.dot(xb, W_ref[...], preferred_element_type=jnp.float32)
    hE = jnp.dot(xb, A_ref[...], preferred_element_type=jnp.float32)
    rep = jnp.dot(c_ref[...], M_ref[...], preferred_element_type=jnp.float32)
    out_ref[...] = (
        acc
        + jnp.dot(
            (hE * rep).astype(jnp.bfloat16),
            B_ref[...],
            preferred_element_type=jnp.float32,
        )
        + b_ref[...]
    )


def kernel(x, gate_w1, gate_b1, gate_w2, base_W, base_b, lora_A, lora_B):
    T, D = x.shape

    logits = pl.pallas_call(
        _gate_body,
        grid=(T // _TB_GATE,),
        in_specs=[
            pl.BlockSpec((_TB_GATE, D), lambda i: (i, 0)),
            pl.BlockSpec((D, _HID), lambda i: (0, 0)),
            pl.BlockSpec((1, _HID), lambda i: (0, 0)),
            pl.BlockSpec((_HID, _NE), lambda i: (0, 0)),
        ],
        out_specs=pl.BlockSpec((_TB_GATE, _NE), lambda i: (i, 0)),
        out_shape=jax.ShapeDtypeStruct((T, _NE), jnp.float32),
        compiler_params=pltpu.CompilerParams(dimension_semantics=("parallel",)),
    )(
        x,
        gate_w1.astype(jnp.bfloat16),
        gate_b1.reshape(1, _HID),
        gate_w2.astype(jnp.bfloat16),
    )

    rows = T // _SC_WORKERS
    route = pl.kernel(
        _route_body,
        out_type=jax.ShapeDtypeStruct((T, _NE), jnp.float32),
        mesh=plsc.VectorSubcoreMesh(core_axis_name="c", subcore_axis_name="s"),
        scratch_types=[
            pltpu.VMEM((rows, _NE), jnp.float32),
            pltpu.VMEM((rows, _NE), jnp.float32),
        ],
        compiler_params=pltpu.CompilerParams(needs_layout_passes=False),
    )
    combine = route(logits)

    A2 = lora_A.transpose(1, 0, 2).reshape(D, _NE * _RK).astype(jnp.bfloat16)
    B2 = lora_B.reshape(_NE * _RK, D).astype(jnp.bfloat16)
    expand = jnp.kron(
        jnp.eye(_NE, dtype=jnp.float32), jnp.ones((1, _RK), jnp.float32)
    )

    out = pl.pallas_call(
        _main_body,
        grid=(T // _TB_MAIN,),
        in_specs=[
            pl.BlockSpec((_TB_MAIN, D), lambda i: (i, 0)),
            pl.BlockSpec((D, D), lambda i: (0, 0)),
            pl.BlockSpec((1, D), lambda i: (0, 0)),
            pl.BlockSpec((D, _NE * _RK), lambda i: (0, 0)),
            pl.BlockSpec((_NE * _RK, D), lambda i: (0, 0)),
            pl.BlockSpec((_TB_MAIN, _NE), lambda i: (i, 0)),
            pl.BlockSpec((_NE, _NE * _RK), lambda i: (0, 0)),
        ],
        out_specs=pl.BlockSpec((_TB_MAIN, D), lambda i: (i, 0)),
        out_shape=jax.ShapeDtypeStruct((T, D), jnp.float32),
        compiler_params=pltpu.CompilerParams(dimension_semantics=("parallel",)),
    )(x, base_W.astype(jnp.bfloat16), base_b.reshape(1, D), A2, B2, combine, expand)
    return out


# speedup vs baseline: 1.0376x; 1.0376x over previous
import functools

import jax
import jax.numpy as jnp
from jax import lax
from jax.experimental import pallas as pl
from jax.experimental.pallas import tpu as pltpu
from jax.experimental.pallas import tpu_sc as plsc

_NE = 16
_RK = 16
_HID = 256

_TB_GATE = 2048
_TB_MAIN = 1024

_SC_WORKERS = 32


def _gate_body(x_ref, w1_ref, b1_ref, w2_ref, out_ref):
    xb = x_ref[...].astype(jnp.bfloat16)
    h = jnp.tanh(
        jnp.dot(xb, w1_ref[...], preferred_element_type=jnp.float32) + b1_ref[...]
    )
    out_ref[...] = jnp.dot(
        h.astype(jnp.bfloat16), w2_ref[...], preferred_element_type=jnp.float32
    )


def _route_body(logits_hbm, comb_hbm, in_v, out_v):
    rows = in_v.shape[0]
    wid = lax.axis_index("c") * 16 + lax.axis_index("s")
    base = wid * rows
    pltpu.sync_copy(logits_hbm.at[pl.ds(base, rows)], in_v)

    @pl.loop(0, rows)
    def _(i):
        v = in_v[i]
        idx = lax.iota(jnp.int32, 16)
        m1 = jnp.max(v)
        i1 = jnp.min(jnp.where(v == m1, idx, 16))
        vm = jnp.where(idx == i1, -jnp.inf, v)
        m2 = jnp.max(vm)
        i2 = jnp.min(jnp.where(vm == m2, idx, 16))
        sel = (idx == i1) | (idx == i2)
        ex = jnp.exp(v - m1)
        num = jnp.where(sel, ex, 0.0)
        out_v[i] = num / jnp.sum(num)

    pltpu.sync_copy(out_v, comb_hbm.at[pl.ds(base, rows)])


def _main_body(x_ref, W_ref, b_ref, A_ref, B_ref, c_ref, M_ref, out_ref):
    rep = jnp.dot(c_ref[...], M_ref[...], preferred_element_type=jnp.float32)
    xb = x_ref[...].astype(jnp.bfloat16)
    hE = jnp.dot(xb, A_ref[...], preferred_element_type=jnp.float32)
    wh = (hE * rep).astype(jnp.bfloat16)
    acc = jnp.dot(xb, W_ref[...], preferred_element_type=jnp.float32)
    out_ref[...] = (
        acc
        + jnp.dot(wh, B_ref[...], preferred_element_type=jnp.float32)
        + b_ref[...]
    )


def kernel(x, gate_w1, gate_b1, gate_w2, base_W, base_b, lora_A, lora_B):
    T, D = x.shape

    logits = pl.pallas_call(
        _gate_body,
        grid=(T // _TB_GATE,),
        in_specs=[
            pl.BlockSpec((_TB_GATE, D), lambda i: (i, 0)),
            pl.BlockSpec((D, _HID), lambda i: (0, 0)),
            pl.BlockSpec((1, _HID), lambda i: (0, 0)),
            pl.BlockSpec((_HID, _NE), lambda i: (0, 0)),
        ],
        out_specs=pl.BlockSpec((_TB_GATE, _NE), lambda i: (i, 0)),
        out_shape=jax.ShapeDtypeStruct((T, _NE), jnp.float32),
        compiler_params=pltpu.CompilerParams(dimension_semantics=("parallel",)),
    )(
        x,
        gate_w1.astype(jnp.bfloat16),
        gate_b1.reshape(1, _HID),
        gate_w2.astype(jnp.bfloat16),
    )

    rows = T // _SC_WORKERS
    route = pl.kernel(
        _route_body,
        out_type=jax.ShapeDtypeStruct((T, _NE), jnp.float32),
        mesh=plsc.VectorSubcoreMesh(core_axis_name="c", subcore_axis_name="s"),
        scratch_types=[
            pltpu.VMEM((rows, _NE), jnp.float32),
            pltpu.VMEM((rows, _NE), jnp.float32),
        ],
        compiler_params=pltpu.CompilerParams(needs_layout_passes=False),
    )
    combine = route(logits)

    A2 = lora_A.transpose(1, 0, 2).reshape(D, _NE * _RK).astype(jnp.bfloat16)
    B2 = lora_B.reshape(_NE * _RK, D).astype(jnp.bfloat16)
    expand = jnp.kron(
        jnp.eye(_NE, dtype=jnp.float32), jnp.ones((1, _RK), jnp.float32)
    )

    out = pl.pallas_call(
        _main_body,
        grid=(T // _TB_MAIN,),
        in_specs=[
            pl.BlockSpec((_TB_MAIN, D), lambda i: (i, 0)),
            pl.BlockSpec((D, D), lambda i: (0, 0)),
            pl.BlockSpec((1, D), lambda i: (0, 0)),
            pl.BlockSpec((D, _NE * _RK), lambda i: (0, 0)),
            pl.BlockSpec((_NE * _RK, D), lambda i: (0, 0)),
            pl.BlockSpec((_TB_MAIN, _NE), lambda i: (i, 0)),
            pl.BlockSpec((_NE, _NE * _RK), lambda i: (0, 0)),
        ],
        out_specs=pl.BlockSpec((_TB_MAIN, D), lambda i: (i, 0)),
        out_shape=jax.ShapeDtypeStruct((T, D), jnp.float32),
        compiler_params=pltpu.CompilerParams(dimension_semantics=("parallel",)),
    )(x, base_W.astype(jnp.bfloat16), base_b.reshape(1, D), A2, B2, combine, expand)
    return out
